# baseline (speedup 1.0000x reference)
import functools

import jax
import jax.numpy as jnp
from jax.experimental import pallas as pl
from jax.experimental.pallas import tpu as pltpu
from jax.experimental.pallas import tpu_sc as plsc

N_CLASSES = 64
N_PER = 32
N_SUPPORT = 8
D = 256
N = N_CLASSES * N_PER
N_QUERY = N_PER - N_SUPPORT
_LANES = 16
_N_WORKERS = 32


_N_CORES = 1
_N_SUB = 16
_PER_W = N_CLASSES // (_N_CORES * _N_SUB)


def _sc_prototypes(x):
    mesh = plsc.VectorSubcoreMesh(core_axis_name="c", subcore_axis_name="s",
                                  num_cores=_N_CORES)

    @functools.partial(
        pl.kernel,
        out_type=jax.ShapeDtypeStruct((N_CLASSES, D), jnp.float32),
        mesh=mesh,
        scratch_types=[
            pltpu.VMEM((_PER_W * N_SUPPORT, D), jnp.float32),
            pltpu.VMEM((_PER_W, D), jnp.float32),
            [pltpu.SemaphoreType.DMA] * _PER_W,
            pltpu.SemaphoreType.DMA,
        ],
    )
    def proto_kernel(x_hbm, p_hbm, buf, rows, sems_in, sem_out):
        cid = jax.lax.axis_index("c")
        sid = jax.lax.axis_index("s")
        wid = sid * _N_CORES + cid
        copies = []
        for j in range(_PER_W):
            cls = wid + _N_CORES * _N_SUB * j
            copies.append(pltpu.async_copy(
                x_hbm.at[pl.ds(cls * N_PER, N_SUPPORT)],
                buf.at[pl.ds(j * N_SUPPORT, N_SUPPORT)], sems_in[j]))

        outs = []
        for j in range(_PER_W):
            copies[j].wait()

            def chunk(k, carry, j=j):
                sl = pl.ds(k * _LANES, _LANES)
                acc = buf[j * N_SUPPORT, sl]
                for r in range(1, N_SUPPORT):
                    acc = acc + buf[j * N_SUPPORT + r, sl]
                rows[j, sl] = acc * (1.0 / N_SUPPORT)
                return carry

            jax.lax.fori_loop(0, D // _LANES, chunk, 0)
            cls = wid + _N_CORES * _N_SUB * j
            outs.append(pltpu.async_copy(rows.at[j], p_hbm.at[cls], sem_out))
        for cp in outs:
            cp.wait()

    return proto_kernel(x)


def _loss_tc_kernel(x_ref, p_ref, out_ref):
    x = x_ref[...]
    p = p_ref[...]

    pn_row = jax.lax.dot_general(
        jnp.ones((1, D), jnp.float32), p * p,
        (((1,), (1,)), ((), ())), preferred_element_type=jnp.float32)

    g = jax.lax.dot_general(
        x, p, (((1,), (1,)), ((), ())),
        preferred_element_type=jnp.float32)
    a = 2.0 * g - pn_row

    m = jnp.max(a, axis=1, keepdims=True)
    lse = m + jnp.log(jnp.sum(jnp.exp(a - m), axis=1, keepdims=True))

    ri = jax.lax.broadcasted_iota(jnp.int32, (N, N_CLASSES), 0)
    ci = jax.lax.broadcasted_iota(jnp.int32, (N, N_CLASSES), 1)
    pick = jnp.sum(jnp.where(ci == ri // N_PER, a, 0.0), axis=1, keepdims=True)

    per_row = lse - pick
    row = jax.lax.broadcasted_iota(jnp.int32, (N, 1), 0)
    is_query = (row % N_PER) >= N_SUPPORT
    total = jnp.sum(jnp.where(is_query, per_row, 0.0), axis=0, keepdims=True)
    out_ref[...] = total / float(N_CLASSES * N_QUERY)


def kernel(input, target):
    del target
    p = _sc_prototypes(input)
    out = pl.pallas_call(
        _loss_tc_kernel,
        out_shape=jax.ShapeDtypeStruct((1, 1), jnp.float32),
    )(input, p)
    return out.reshape(())


# speedup vs baseline: 1.0176x; 1.0176x over previous
import functools

import jax
import jax.numpy as jnp
from jax.experimental import pallas as pl
from jax.experimental.pallas import tpu as pltpu
from jax.experimental.pallas import tpu_sc as plsc

N_CLASSES = 64
N_PER = 32
N_SUPPORT = 8
D = 256
N = N_CLASSES * N_PER
N_QUERY = N_PER - N_SUPPORT
_LANES = 16
_N_WORKERS = 32


_N_CORES = 1
_N_SUB = 16
_PER_W = N_CLASSES // (_N_CORES * _N_SUB)


def _sc_prototypes(x):
    mesh = plsc.VectorSubcoreMesh(core_axis_name="c", subcore_axis_name="s",
                                  num_cores=_N_CORES)

    @functools.partial(
        pl.kernel,
        out_type=jax.ShapeDtypeStruct((N_CLASSES, D), jnp.float32),
        mesh=mesh,
        scratch_types=[
            pltpu.VMEM((_PER_W * N_SUPPORT, D), jnp.float32),
            pltpu.VMEM((_PER_W, D), jnp.float32),
            pltpu.SemaphoreType.DMA,
            pltpu.SemaphoreType.DMA,
        ],
    )
    def proto_kernel(x_hbm, p_hbm, buf, rows, sem_in, sem_out):
        cid = jax.lax.axis_index("c")
        sid = jax.lax.axis_index("s")
        wid = sid * _N_CORES + cid
        copies = []
        for j in range(_PER_W):
            cls = wid + _N_CORES * _N_SUB * j
            copies.append(pltpu.async_copy(
                x_hbm.at[pl.ds(cls * N_PER, N_SUPPORT)],
                buf.at[pl.ds(j * N_SUPPORT, N_SUPPORT)], sem_in))
        for cp in copies:
            cp.wait()

        def chunk(k, carry):
            sl = pl.ds(k * _LANES, _LANES)
            for j in range(_PER_W):
                acc = buf[j * N_SUPPORT, sl]
                for r in range(1, N_SUPPORT):
                    acc = acc + buf[j * N_SUPPORT + r, sl]
                rows[j, sl] = acc * (1.0 / N_SUPPORT)
            return carry

        jax.lax.fori_loop(0, D // _LANES, chunk, 0)
        outs = []
        for j in range(_PER_W):
            cls = wid + _N_CORES * _N_SUB * j
            outs.append(pltpu.async_copy(rows.at[j], p_hbm.at[cls], sem_out))
        for cp in outs:
            cp.wait()

    return proto_kernel(x)


def _loss_tc_kernel(x_ref, p_ref, out_ref):
    x = x_ref[...]
    p = p_ref[...]

    pn_row = jax.lax.dot_general(
        jnp.ones((1, D), jnp.float32), p * p,
        (((1,), (1,)), ((), ())), preferred_element_type=jnp.float32)

    g = jax.lax.dot_general(
        x, p, (((1,), (1,)), ((), ())),
        preferred_element_type=jnp.float32)
    a = 2.0 * g - pn_row

    m = jnp.max(a, axis=1, keepdims=True)
    lse = m + jnp.log(jnp.sum(jnp.exp(a - m), axis=1, keepdims=True))

    ri = jax.lax.broadcasted_iota(jnp.int32, (N, N_CLASSES), 0)
    ci = jax.lax.broadcasted_iota(jnp.int32, (N, N_CLASSES), 1)
    pick = jnp.sum(jnp.where(ci == ri // N_PER, a, 0.0), axis=1, keepdims=True)

    per_row = lse - pick
    row = jax.lax.broadcasted_iota(jnp.int32, (N, 1), 0)
    is_query = (row % N_PER) >= N_SUPPORT
    total = jnp.sum(jnp.where(is_query, per_row, 0.0), axis=0, keepdims=True)
    out_ref[...] = total / float(N_CLASSES * N_QUERY)


def kernel(input, target):
    del target
    p = _sc_prototypes(input)
    out = pl.pallas_call(
        _loss_tc_kernel,
        out_shape=jax.ShapeDtypeStruct((1, 1), jnp.float32),
    )(input, p)
    return out.reshape(())
